# baseline (speedup 1.0000x reference)
import functools

import jax
import jax.numpy as jnp
from jax import lax
from jax.experimental import pallas as pl
from jax.experimental.pallas import tpu as pltpu
from jax.experimental.pallas import tpu_sc as plsc

N = 10000
E = 320000
D = 128
NC = 2
NS = 16
NW = NC * NS
NPAD = 10240
EPT = E // NW
CH = 40
NCH = EPT // CH
SCAT = 80


def _prep_body(h_ref, wfc_ref, wa_ref, z_ref, sl_ref, sr_ref):
    h = h_ref[...]
    z = lax.dot_general(h, wfc_ref[...], (((1,), (1,)), ((), ())),
                        preferred_element_type=jnp.float32)
    z_ref[...] = z
    a2 = wa_ref[...].reshape(2, D)
    s2 = lax.dot_general(z, a2, (((1,), (1,)), ((), ())),
                         preferred_element_type=jnp.float32)
    sl_ref[...] = s2[:, 0:1]
    sr_ref[...] = s2[:, 1:2]


def _add_body(a_ref, b_ref, o_ref):
    o_ref[...] = a_ref[...] + b_ref[...]


_sc_mesh = plsc.VectorSubcoreMesh(
    core_axis_name="c", subcore_axis_name="s", num_cores=NC, num_subcores=NS)


@functools.partial(
    pl.kernel,
    out_type=jax.ShapeDtypeStruct((NC, N, D), jnp.float32),
    mesh=_sc_mesh,
    compiler_params=pltpu.CompilerParams(
        needs_layout_passes=False, use_tc_tiling_on_sc=False),
    scratch_types=[
        pltpu.VMEM_SHARED((N, D), jnp.float32),
        pltpu.VMEM_SHARED((NPAD,), jnp.float32),
    ],
)
def _sc_gat(srcA_hbm, dstA_hbm, sl_hbm, sr_hbm, z_hbm, out_hbm,
            hacc_sh, den_sh):
    c = lax.axis_index("c")
    s = lax.axis_index("s")
    pl.run_scoped(
        functools.partial(_sc_gat_body, srcA_hbm, dstA_hbm, sl_hbm, sr_hbm,
                          z_hbm, out_hbm, hacc_sh, den_sh, c, s),
        pltpu.VMEM((EPT,), jnp.int32),
        pltpu.VMEM((EPT,), jnp.int32),
        pltpu.VMEM((EPT,), jnp.float32),
        pltpu.SemaphoreType.DMA,
    )


def _sc_gat_body(srcA_hbm, dstA_hbm, sl_hbm, sr_hbm, z_hbm, out_hbm,
                 hacc_sh, den_sh, c, s, src_v, dst_v, ex_v, dsem):

    def phase1(sl_v, sr_v):
        pltpu.sync_copy(sl_hbm, sl_v)
        pltpu.sync_copy(sr_hbm, sr_v)

        for k in range(640 // 16):
            ex_v[pl.ds(k * 16, 16)] = jnp.zeros((16,), jnp.float32)
        pltpu.sync_copy(ex_v.at[pl.ds(0, 640)], den_sh.at[pl.ds(s * 640, 640)])

        def lane_max(acc):
            dnums = lax.GatherDimensionNumbers(
                offset_dims=(), collapsed_slice_dims=(0,), start_index_map=(0,))
            for sh in (8, 4, 2, 1):
                idx = lax.iota(jnp.int32, 16) ^ sh
                perm = lax.gather(
                    acc, idx[:, None], dnums, slice_sizes=(1,),
                    mode=lax.GatherScatterMode.PROMISE_IN_BOUNDS)
                acc = jnp.maximum(acc, perm)
            return acc

        def table_max(tab):
            def body(i, acc):
                return jnp.maximum(acc, tab[pl.ds(i * 16, 16)])
            return lane_max(lax.fori_loop(
                0, N // 16, body, jnp.full((16,), -jnp.inf, jnp.float32)))

        cv = table_max(sl_v) + table_max(sr_v)
        cv = jnp.maximum(cv, 0.01 * cv)

        plsc.subcore_barrier()

        @pl.loop(0, NC)
        def _half(q):
            b = 1 - c + q * (2 * c - 1)
            pltpu.sync_copy(srcA_hbm.at[s].at[b], src_v)
            pltpu.sync_copy(dstA_hbm.at[s].at[b], dst_v)

            @pl.loop(0, EPT // 16)
            def _edge(i):
                sidx = src_v[pl.ds(i * 16, 16)]
                didx = dst_v[pl.ds(i * 16, 16)]
                x = (plsc.load_gather(sl_v, [sidx])
                     + plsc.load_gather(sr_v, [didx]))
                e = jnp.maximum(x, 0.01 * x)
                ex_v[pl.ds(i * 16, 16)] = jnp.exp(e - cv)

            @pl.loop(0, EPT // SCAT)
            def _sc(j):
                pltpu.async_copy(
                    ex_v.at[pl.ds(j * SCAT, SCAT)],
                    den_sh.at[dst_v.at[pl.ds(j * SCAT, SCAT)]],
                    dsem, add=True)

            @pl.loop(0, EPT // SCAT)
            def _dr(j):
                pltpu.make_async_copy(
                    ex_v.at[pl.ds(0, SCAT)],
                    den_sh.at[dst_v.at[pl.ds(0, SCAT)]], dsem).wait()

        plsc.subcore_barrier()

    with jax.named_scope("p1_denom"):
        pl.run_scoped(
            phase1,
            pltpu.VMEM((N,), jnp.float32),
            pltpu.VMEM((N,), jnp.float32),
        )

    def phase2_prep(rcp_v):
        pltpu.sync_copy(den_sh.at[pl.ds(0, N)], rcp_v)

        @pl.loop(0, N // 16)
        def _rcp(i):
            v = rcp_v[pl.ds(i * 16, 16)]
            rcp_v[pl.ds(i * 16, 16)] = jnp.where(v == 0.0, 1.0, 1.0 / v)

        @pl.loop(0, EPT // 16)
        def _al(i):
            didx = dst_v[pl.ds(i * 16, 16)]
            r16 = plsc.load_gather(rcp_v, [didx])
            ex_v[pl.ds(i * 16, 16)] = ex_v[pl.ds(i * 16, 16)] * r16

    with jax.named_scope("p2_alpha"):
        pl.run_scoped(phase2_prep, pltpu.VMEM((N,), jnp.float32))

    def phase2(bufA, bufB, gsA, gsB, ssA, ssB):
        @pl.loop(0, CH)
        def _zr(r):
            for t in range(D // 16):
                bufA[r, pl.ds(t * 16, 16)] = jnp.zeros((16,), jnp.float32)

        nzh = jnp.where(s == NS - 1, 10, 16)

        @pl.loop(0, nzh)
        def _zh(q):
            pltpu.sync_copy(bufA, hacc_sh.at[pl.ds(s * 640 + q * CH, CH)])

        plsc.subcore_barrier()

        def gather(t, buf, sem):
            pltpu.async_copy(
                z_hbm.at[src_v.at[pl.ds(t * CH, CH)]], buf, sem)

        def scat(t, buf, sem):
            pltpu.async_copy(
                buf, hacc_sh.at[dst_v.at[pl.ds(t * CH, CH)]], sem, add=True)

        def wait_gather(t, buf, sem):
            pltpu.make_async_copy(
                z_hbm.at[src_v.at[pl.ds(t * CH, CH)]], buf, sem).wait()

        def wait_scat(t, buf, sem):
            pltpu.make_async_copy(
                buf, hacc_sh.at[dst_v.at[pl.ds(t * CH, CH)]], sem).wait()

        def scale(t, buf):
            @pl.loop(0, CH)
            def _row(r):
                av = plsc.load_gather(
                    ex_v, [jnp.full((16,), t * CH + r, jnp.int32)])
                for u in range(D // 16):
                    buf[r, pl.ds(u * 16, 16)] = buf[r, pl.ds(u * 16, 16)] * av

        def chunk(t, cur, oth, gs_cur, gs_oth, ss_cur, ss_oth):
            wait_gather(t, cur, gs_cur)
            scale(t, cur)

            @pl.when(t >= 1)
            def _drain():
                wait_scat(t - 1, oth, ss_oth)

            @pl.when(t + 1 < NCH)
            def _pref():
                gather(t + 1, oth, gs_oth)

            scat(t, cur, ss_cur)

        gather(0, bufA, gsA)

        @pl.loop(0, NCH // 2)
        def _pair(i):
            chunk(2 * i, bufA, bufB, gsA, gsB, ssA, ssB)
            chunk(2 * i + 1, bufB, bufA, gsB, gsA, ssB, ssA)

        wait_scat(NCH - 1, bufB, ssB)

        plsc.subcore_barrier()

        nch = jnp.where(s == NS - 1, 10, 16)

        @pl.loop(0, nch)
        def _wb(q):
            r0 = s * 640 + q * CH
            pltpu.sync_copy(hacc_sh.at[pl.ds(r0, CH)], bufA)
            pltpu.sync_copy(bufA, out_hbm.at[c].at[pl.ds(r0, CH)])

    with jax.named_scope("p2_main"):
        pl.run_scoped(
            phase2,
            pltpu.VMEM((CH, D), jnp.float32),
            pltpu.VMEM((CH, D), jnp.float32),
            pltpu.SemaphoreType.DMA,
            pltpu.SemaphoreType.DMA,
            pltpu.SemaphoreType.DMA,
            pltpu.SemaphoreType.DMA,
        )


def kernel(h, edge_index, W_fc, W_attn):
    src = edge_index[0].astype(jnp.int32)
    dst = edge_index[1].astype(jnp.int32)
    srcA = src.reshape(NS, NC, EPT)
    dstA = dst.reshape(NS, NC, EPT)

    z, sl, sr = pl.pallas_call(
        _prep_body,
        grid=(10,),
        in_specs=[
            pl.BlockSpec((N // 10, D), lambda i: (i, 0)),
            pl.BlockSpec((D, D), lambda i: (0, 0)),
            pl.BlockSpec((1, 2 * D), lambda i: (0, 0)),
        ],
        out_specs=[
            pl.BlockSpec((N // 10, D), lambda i: (i, 0)),
            pl.BlockSpec((N // 10, 1), lambda i: (i, 0)),
            pl.BlockSpec((N // 10, 1), lambda i: (i, 0)),
        ],
        out_shape=[
            jax.ShapeDtypeStruct((N, D), jnp.float32),
            jax.ShapeDtypeStruct((N, 1), jnp.float32),
            jax.ShapeDtypeStruct((N, 1), jnp.float32),
        ],
    )(h, W_fc, W_attn)

    hpart = _sc_gat(srcA, dstA, sl.reshape(N), sr.reshape(N), z)

    h_out = pl.pallas_call(
        _add_body,
        grid=(10,),
        in_specs=[
            pl.BlockSpec((N // 10, D), lambda i: (i, 0)),
            pl.BlockSpec((N // 10, D), lambda i: (i, 0)),
        ],
        out_specs=pl.BlockSpec((N // 10, D), lambda i: (i, 0)),
        out_shape=jax.ShapeDtypeStruct((N, D), jnp.float32),
    )(hpart[0], hpart[1])
    return h_out


# speedup vs baseline: 1.0111x; 1.0111x over previous
import functools

import jax
import jax.numpy as jnp
from jax import lax
from jax.experimental import pallas as pl
from jax.experimental.pallas import tpu as pltpu
from jax.experimental.pallas import tpu_sc as plsc

N = 10000
E = 320000
D = 128
NC = 2
NS = 16
NW = NC * NS
NPAD = 10240
EPT = E // NW
CH = 40
NCH = EPT // CH
SCAT = 80


def _prep_body(h_ref, wfc_ref, wa_ref, z_ref, sl_ref, sr_ref):
    h = h_ref[...]
    z = lax.dot_general(h, wfc_ref[...], (((1,), (1,)), ((), ())),
                        preferred_element_type=jnp.float32)
    z_ref[...] = z
    a2 = wa_ref[...].reshape(2, D)
    s2 = lax.dot_general(z, a2, (((1,), (1,)), ((), ())),
                         preferred_element_type=jnp.float32)
    sl_ref[...] = s2[:, 0:1]
    sr_ref[...] = s2[:, 1:2]


def _add_body(a_ref, b_ref, o_ref):
    o_ref[...] = a_ref[...] + b_ref[...]


_sc_mesh = plsc.VectorSubcoreMesh(
    core_axis_name="c", subcore_axis_name="s", num_cores=NC, num_subcores=NS)


@functools.partial(
    pl.kernel,
    out_type=jax.ShapeDtypeStruct((NC, N, D), jnp.float32),
    mesh=_sc_mesh,
    compiler_params=pltpu.CompilerParams(
        needs_layout_passes=False, use_tc_tiling_on_sc=False),
    scratch_types=[
        pltpu.VMEM_SHARED((N, D), jnp.float32),
        pltpu.VMEM_SHARED((NPAD,), jnp.float32),
    ],
)
def _sc_gat(srcA_hbm, dstA_hbm, sl_hbm, sr_hbm, z_hbm, out_hbm,
            hacc_sh, den_sh):
    c = lax.axis_index("c")
    s = lax.axis_index("s")
    pl.run_scoped(
        functools.partial(_sc_gat_body, srcA_hbm, dstA_hbm, sl_hbm, sr_hbm,
                          z_hbm, out_hbm, hacc_sh, den_sh, c, s),
        pltpu.VMEM((EPT,), jnp.int32),
        pltpu.VMEM((EPT,), jnp.int32),
        pltpu.VMEM((EPT + 16,), jnp.float32),
        pltpu.SemaphoreType.DMA,
    )


def _sc_gat_body(srcA_hbm, dstA_hbm, sl_hbm, sr_hbm, z_hbm, out_hbm,
                 hacc_sh, den_sh, c, s, src_v, dst_v, ex_v, dsem):

    def phase1(sl_v, sr_v):
        pltpu.sync_copy(sl_hbm, sl_v)
        pltpu.sync_copy(sr_hbm, sr_v)

        for k in range(640 // 16):
            ex_v[pl.ds(k * 16, 16)] = jnp.zeros((16,), jnp.float32)
        pltpu.sync_copy(ex_v.at[pl.ds(0, 640)], den_sh.at[pl.ds(s * 640, 640)])

        def lane_max(acc):
            dnums = lax.GatherDimensionNumbers(
                offset_dims=(), collapsed_slice_dims=(0,), start_index_map=(0,))
            for sh in (8, 4, 2, 1):
                idx = lax.iota(jnp.int32, 16) ^ sh
                perm = lax.gather(
                    acc, idx[:, None], dnums, slice_sizes=(1,),
                    mode=lax.GatherScatterMode.PROMISE_IN_BOUNDS)
                acc = jnp.maximum(acc, perm)
            return acc

        def table_max(tab):
            def body(i, acc):
                return jnp.maximum(acc, tab[pl.ds(i * 16, 16)])
            return lane_max(lax.fori_loop(
                0, N // 16, body, jnp.full((16,), -jnp.inf, jnp.float32)))

        cv = table_max(sl_v) + table_max(sr_v)
        cv = jnp.maximum(cv, 0.01 * cv)

        plsc.subcore_barrier()

        @pl.loop(0, NC)
        def _half(q):
            b = 1 - c + q * (2 * c - 1)
            pltpu.sync_copy(srcA_hbm.at[s].at[b], src_v)
            pltpu.sync_copy(dstA_hbm.at[s].at[b], dst_v)

            @pl.loop(0, EPT // 16)
            def _edge(i):
                sidx = src_v[pl.ds(i * 16, 16)]
                didx = dst_v[pl.ds(i * 16, 16)]
                x = (plsc.load_gather(sl_v, [sidx])
                     + plsc.load_gather(sr_v, [didx]))
                e = jnp.maximum(x, 0.01 * x)
                ex_v[pl.ds(i * 16, 16)] = jnp.exp(e - cv)

            @pl.loop(0, EPT // SCAT)
            def _sc(j):
                pltpu.async_copy(
                    ex_v.at[pl.ds(j * SCAT, SCAT)],
                    den_sh.at[dst_v.at[pl.ds(j * SCAT, SCAT)]],
                    dsem, add=True)

            @pl.loop(0, EPT // SCAT)
            def _dr(j):
                pltpu.make_async_copy(
                    ex_v.at[pl.ds(0, SCAT)],
                    den_sh.at[dst_v.at[pl.ds(0, SCAT)]], dsem).wait()

        plsc.subcore_barrier()

    with jax.named_scope("p1_denom"):
        pl.run_scoped(
            phase1,
            pltpu.VMEM((N,), jnp.float32),
            pltpu.VMEM((N,), jnp.float32),
        )

    def phase2_prep(rcp_v):
        pltpu.sync_copy(den_sh.at[pl.ds(0, N)], rcp_v)

        @pl.loop(0, N // 16)
        def _rcp(i):
            v = rcp_v[pl.ds(i * 16, 16)]
            rcp_v[pl.ds(i * 16, 16)] = jnp.where(v == 0.0, 1.0, 1.0 / v)

        @pl.loop(0, EPT // 16)
        def _al(i):
            didx = dst_v[pl.ds(i * 16, 16)]
            r16 = plsc.load_gather(rcp_v, [didx])
            ex_v[pl.ds(i * 16, 16)] = ex_v[pl.ds(i * 16, 16)] * r16

    with jax.named_scope("p2_alpha"):
        pl.run_scoped(phase2_prep, pltpu.VMEM((N,), jnp.float32))

    def phase2(bufA, bufB, gsA, gsB, ssA, ssB):
        @pl.loop(0, CH)
        def _zr(r):
            for t in range(D // 16):
                bufA[r, pl.ds(t * 16, 16)] = jnp.zeros((16,), jnp.float32)

        nzh = jnp.where(s == NS - 1, 10, 16)

        @pl.loop(0, nzh)
        def _zh(q):
            pltpu.sync_copy(bufA, hacc_sh.at[pl.ds(s * 640 + q * CH, CH)])

        plsc.subcore_barrier()

        def gather(t, buf, sem):
            pltpu.async_copy(
                z_hbm.at[src_v.at[pl.ds(t * CH, CH)]], buf, sem)

        def scat(t, buf, sem):
            pltpu.async_copy(
                buf, hacc_sh.at[dst_v.at[pl.ds(t * CH, CH)]], sem, add=True)

        def wait_gather(t, buf, sem):
            pltpu.make_async_copy(
                z_hbm.at[src_v.at[pl.ds(t * CH, CH)]], buf, sem).wait()

        def wait_scat(t, buf, sem):
            pltpu.make_async_copy(
                buf, hacc_sh.at[dst_v.at[pl.ds(t * CH, CH)]], sem).wait()

        dnums = lax.GatherDimensionNumbers(
            offset_dims=(), collapsed_slice_dims=(0,), start_index_map=(0,))

        def scale(t, buf):
            @pl.loop(0, CH // 8)
            def _grp(p):
                a16 = ex_v[pl.ds(t * CH + p * 8, 16)]
                for r in range(8):
                    av = lax.gather(
                        a16, jnp.full((16, 1), r, jnp.int32), dnums,
                        slice_sizes=(1,),
                        mode=lax.GatherScatterMode.PROMISE_IN_BOUNDS)
                    row = p * 8 + r
                    for u in range(D // 16):
                        buf[row, pl.ds(u * 16, 16)] = (
                            buf[row, pl.ds(u * 16, 16)] * av)

        def chunk(t, cur, oth, gs_cur, gs_oth, ss_cur, ss_oth):
            wait_gather(t, cur, gs_cur)
            scale(t, cur)

            @pl.when(t >= 1)
            def _drain():
                wait_scat(t - 1, oth, ss_oth)

            @pl.when(t + 1 < NCH)
            def _pref():
                gather(t + 1, oth, gs_oth)

            scat(t, cur, ss_cur)

        gather(0, bufA, gsA)

        @pl.loop(0, NCH // 2)
        def _pair(i):
            chunk(2 * i, bufA, bufB, gsA, gsB, ssA, ssB)
            chunk(2 * i + 1, bufB, bufA, gsB, gsA, ssB, ssA)

        wait_scat(NCH - 1, bufB, ssB)

        plsc.subcore_barrier()

        nch = jnp.where(s == NS - 1, 10, 16)

        @pl.loop(0, nch)
        def _wb(q):
            r0 = s * 640 + q * CH
            pltpu.sync_copy(hacc_sh.at[pl.ds(r0, CH)], bufA)
            pltpu.sync_copy(bufA, out_hbm.at[c].at[pl.ds(r0, CH)])

    with jax.named_scope("p2_main"):
        pl.run_scoped(
            phase2,
            pltpu.VMEM((CH, D), jnp.float32),
            pltpu.VMEM((CH, D), jnp.float32),
            pltpu.SemaphoreType.DMA,
            pltpu.SemaphoreType.DMA,
            pltpu.SemaphoreType.DMA,
            pltpu.SemaphoreType.DMA,
        )


def kernel(h, edge_index, W_fc, W_attn):
    src = edge_index[0].astype(jnp.int32)
    dst = edge_index[1].astype(jnp.int32)
    srcA = src.reshape(NS, NC, EPT)
    dstA = dst.reshape(NS, NC, EPT)

    z, sl, sr = pl.pallas_call(
        _prep_body,
        grid=(10,),
        in_specs=[
            pl.BlockSpec((N // 10, D), lambda i: (i, 0)),
            pl.BlockSpec((D, D), lambda i: (0, 0)),
            pl.BlockSpec((1, 2 * D), lambda i: (0, 0)),
        ],
        out_specs=[
            pl.BlockSpec((N // 10, D), lambda i: (i, 0)),
            pl.BlockSpec((N // 10, 1), lambda i: (i, 0)),
            pl.BlockSpec((N // 10, 1), lambda i: (i, 0)),
        ],
        out_shape=[
            jax.ShapeDtypeStruct((N, D), jnp.float32),
            jax.ShapeDtypeStruct((N, 1), jnp.float32),
            jax.ShapeDtypeStruct((N, 1), jnp.float32),
        ],
    )(h, W_fc, W_attn)

    hpart = _sc_gat(srcA, dstA, sl.reshape(N), sr.reshape(N), z)

    h_out = pl.pallas_call(
        _add_body,
        grid=(10,),
        in_specs=[
            pl.BlockSpec((N // 10, D), lambda i: (i, 0)),
            pl.BlockSpec((N // 10, D), lambda i: (i, 0)),
        ],
        out_specs=pl.BlockSpec((N // 10, D), lambda i: (i, 0)),
        out_shape=jax.ShapeDtypeStruct((N, D), jnp.float32),
    )(hpart[0], hpart[1])
    return h_out


# speedup vs baseline: 1.0111x; 1.0000x over previous
import functools

import jax
import jax.numpy as jnp
from jax import lax
from jax.experimental import pallas as pl
from jax.experimental.pallas import tpu as pltpu
from jax.experimental.pallas import tpu_sc as plsc

N = 10000
E = 320000
D = 128
NC = 2
NS = 16
NW = NC * NS
NPAD = 10240
EPT = E // NW
CH = 40
NCH = EPT // CH
SCAT = 80


def _prep_body(h_ref, wfc_ref, wa_ref, z_ref, sl_ref, sr_ref):
    h = h_ref[...]
    z = lax.dot_general(h, wfc_ref[...], (((1,), (1,)), ((), ())),
                        preferred_element_type=jnp.float32)
    z_ref[...] = z
    a2 = wa_ref[...].reshape(2, D)
    s2 = lax.dot_general(z, a2, (((1,), (1,)), ((), ())),
                         preferred_element_type=jnp.float32)
    sl_ref[...] = s2[:, 0:1]
    sr_ref[...] = s2[:, 1:2]


def _add_body(a_ref, b_ref, o_ref):
    o_ref[...] = a_ref[...] + b_ref[...]


_sc_mesh = plsc.VectorSubcoreMesh(
    core_axis_name="c", subcore_axis_name="s", num_cores=NC, num_subcores=NS)


@functools.partial(
    pl.kernel,
    out_type=jax.ShapeDtypeStruct((NC, N, D), jnp.float32),
    mesh=_sc_mesh,
    compiler_params=pltpu.CompilerParams(
        needs_layout_passes=False, use_tc_tiling_on_sc=False),
    scratch_types=[
        pltpu.VMEM_SHARED((N, D), jnp.float32),
        pltpu.VMEM_SHARED((NPAD,), jnp.float32),
    ],
)
def _sc_gat(srcA_hbm, dstA_hbm, sl_hbm, sr_hbm, z_hbm, out_hbm,
            hacc_sh, den_sh):
    c = lax.axis_index("c")
    s = lax.axis_index("s")
    pl.run_scoped(
        functools.partial(_sc_gat_body, srcA_hbm, dstA_hbm, sl_hbm, sr_hbm,
                          z_hbm, out_hbm, hacc_sh, den_sh, c, s),
        pltpu.VMEM((EPT,), jnp.int32),
        pltpu.VMEM((EPT,), jnp.int32),
        pltpu.VMEM((EPT + 16,), jnp.float32),
        pltpu.SemaphoreType.DMA,
    )


def _sc_gat_body(srcA_hbm, dstA_hbm, sl_hbm, sr_hbm, z_hbm, out_hbm,
                 hacc_sh, den_sh, c, s, src_v, dst_v, ex_v, dsem):

    def phase1(sl_v, sr_v):
        pltpu.sync_copy(sl_hbm, sl_v)
        pltpu.sync_copy(sr_hbm, sr_v)

        for k in range(640 // 16):
            ex_v[pl.ds(k * 16, 16)] = jnp.zeros((16,), jnp.float32)
        pltpu.sync_copy(ex_v.at[pl.ds(0, 640)], den_sh.at[pl.ds(s * 640, 640)])

        def lane_max(acc):
            dnums = lax.GatherDimensionNumbers(
                offset_dims=(), collapsed_slice_dims=(0,), start_index_map=(0,))
            for sh in (8, 4, 2, 1):
                idx = lax.iota(jnp.int32, 16) ^ sh
                perm = lax.gather(
                    acc, idx[:, None], dnums, slice_sizes=(1,),
                    mode=lax.GatherScatterMode.PROMISE_IN_BOUNDS)
                acc = jnp.maximum(acc, perm)
            return acc

        def table_max(tab):
            def body(i, acc):
                return jnp.maximum(acc, tab[pl.ds(i * 16, 16)])
            return lane_max(lax.fori_loop(
                0, N // 16, body, jnp.full((16,), -jnp.inf, jnp.float32)))

        cv = table_max(sl_v) + table_max(sr_v)
        cv = jnp.maximum(cv, 0.01 * cv)

        plsc.subcore_barrier()

        @pl.loop(0, NC)
        def _half(q):
            b = 1 - c + q * (2 * c - 1)
            pltpu.sync_copy(srcA_hbm.at[s].at[b], src_v)
            pltpu.sync_copy(dstA_hbm.at[s].at[b], dst_v)

            @pl.loop(0, EPT // 16)
            def _edge(i):
                sidx = src_v[pl.ds(i * 16, 16)]
                didx = dst_v[pl.ds(i * 16, 16)]
                x = (plsc.load_gather(sl_v, [sidx])
                     + plsc.load_gather(sr_v, [didx]))
                e = jnp.maximum(x, 0.01 * x)
                ex_v[pl.ds(i * 16, 16)] = jnp.exp(e - cv)

            @pl.loop(0, EPT // SCAT)
            def _sc(j):
                pltpu.async_copy(
                    ex_v.at[pl.ds(j * SCAT, SCAT)],
                    den_sh.at[dst_v.at[pl.ds(j * SCAT, SCAT)]],
                    dsem, add=True)

            @pl.loop(0, EPT // SCAT)
            def _dr(j):
                pltpu.make_async_copy(
                    ex_v.at[pl.ds(0, SCAT)],
                    den_sh.at[dst_v.at[pl.ds(0, SCAT)]], dsem).wait()

        plsc.subcore_barrier()

    with jax.named_scope("p1_denom"):
        pl.run_scoped(
            phase1,
            pltpu.VMEM((N,), jnp.float32),
            pltpu.VMEM((N,), jnp.float32),
        )

    def phase2_prep(rcp_v):
        pltpu.sync_copy(den_sh.at[pl.ds(0, N)], rcp_v)

        @pl.loop(0, N // 16)
        def _rcp(i):
            v = rcp_v[pl.ds(i * 16, 16)]
            rcp_v[pl.ds(i * 16, 16)] = jnp.where(v == 0.0, 1.0, 1.0 / v)

        @pl.loop(0, EPT // 16)
        def _al(i):
            didx = dst_v[pl.ds(i * 16, 16)]
            r16 = plsc.load_gather(rcp_v, [didx])
            ex_v[pl.ds(i * 16, 16)] = ex_v[pl.ds(i * 16, 16)] * r16

    with jax.named_scope("p2_alpha"):
        pl.run_scoped(phase2_prep, pltpu.VMEM((N,), jnp.float32))

    def phase2(bufA, bufB, gsA, gsB, ssA, ssB):
        @pl.loop(0, CH)
        def _zr(r):
            for t in range(D // 16):
                bufA[r, pl.ds(t * 16, 16)] = jnp.zeros((16,), jnp.float32)

        nzh = jnp.where(s == NS - 1, 10, 16)

        @pl.loop(0, nzh)
        def _zh(q):
            pltpu.sync_copy(bufA, hacc_sh.at[pl.ds(s * 640 + q * CH, CH)])

        plsc.subcore_barrier()

        def gather(t, buf, sem):
            for p in range(5):
                pltpu.async_copy(
                    z_hbm.at[src_v.at[pl.ds(t * CH + p * 8, 8)]],
                    buf.at[pl.ds(p * 8, 8)], sem)

        def scat(t, buf, sem):
            pltpu.async_copy(
                buf, hacc_sh.at[dst_v.at[pl.ds(t * CH, CH)]], sem, add=True)

        def wait_gather(t, buf, sem):
            for p in range(5):
                pltpu.make_async_copy(
                    z_hbm.at[src_v.at[pl.ds(t * CH + p * 8, 8)]],
                    buf.at[pl.ds(p * 8, 8)], sem).wait()

        def wait_scat(t, buf, sem):
            pltpu.make_async_copy(
                buf, hacc_sh.at[dst_v.at[pl.ds(t * CH, CH)]], sem).wait()

        dnums = lax.GatherDimensionNumbers(
            offset_dims=(), collapsed_slice_dims=(0,), start_index_map=(0,))

        def scale(t, buf):
            @pl.loop(0, CH // 8)
            def _grp(p):
                a16 = ex_v[pl.ds(t * CH + p * 8, 16)]
                for r in range(8):
                    av = lax.gather(
                        a16, jnp.full((16, 1), r, jnp.int32), dnums,
                        slice_sizes=(1,),
                        mode=lax.GatherScatterMode.PROMISE_IN_BOUNDS)
                    row = p * 8 + r
                    for u in range(D // 16):
                        buf[row, pl.ds(u * 16, 16)] = (
                            buf[row, pl.ds(u * 16, 16)] * av)

        def chunk(t, cur, oth, gs_cur, gs_oth, ss_cur, ss_oth):
            wait_gather(t, cur, gs_cur)
            scale(t, cur)

            @pl.when(t >= 1)
            def _drain():
                wait_scat(t - 1, oth, ss_oth)

            @pl.when(t + 1 < NCH)
            def _pref():
                gather(t + 1, oth, gs_oth)

            scat(t, cur, ss_cur)

        gather(0, bufA, gsA)

        @pl.loop(0, NCH // 2)
        def _pair(i):
            chunk(2 * i, bufA, bufB, gsA, gsB, ssA, ssB)
            chunk(2 * i + 1, bufB, bufA, gsB, gsA, ssB, ssA)

        wait_scat(NCH - 1, bufB, ssB)

        plsc.subcore_barrier()

        nch = jnp.where(s == NS - 1, 10, 16)

        @pl.loop(0, nch)
        def _wb(q):
            r0 = s * 640 + q * CH
            pltpu.sync_copy(hacc_sh.at[pl.ds(r0, CH)], bufA)
            pltpu.sync_copy(bufA, out_hbm.at[c].at[pl.ds(r0, CH)])

    with jax.named_scope("p2_main"):
        pl.run_scoped(
            phase2,
            pltpu.VMEM((CH, D), jnp.float32),
            pltpu.VMEM((CH, D), jnp.float32),
            pltpu.SemaphoreType.DMA,
            pltpu.SemaphoreType.DMA,
            pltpu.SemaphoreType.DMA,
            pltpu.SemaphoreType.DMA,
        )


def kernel(h, edge_index, W_fc, W_attn):
    src = edge_index[0].astype(jnp.int32)
    dst = edge_index[1].astype(jnp.int32)
    srcA = src.reshape(NS, NC, EPT)
    dstA = dst.reshape(NS, NC, EPT)

    z, sl, sr = pl.pallas_call(
        _prep_body,
        grid=(10,),
        in_specs=[
            pl.BlockSpec((N // 10, D), lambda i: (i, 0)),
            pl.BlockSpec((D, D), lambda i: (0, 0)),
            pl.BlockSpec((1, 2 * D), lambda i: (0, 0)),
        ],
        out_specs=[
            pl.BlockSpec((N // 10, D), lambda i: (i, 0)),
            pl.BlockSpec((N // 10, 1), lambda i: (i, 0)),
            pl.BlockSpec((N // 10, 1), lambda i: (i, 0)),
        ],
        out_shape=[
            jax.ShapeDtypeStruct((N, D), jnp.float32),
            jax.ShapeDtypeStruct((N, 1), jnp.float32),
            jax.ShapeDtypeStruct((N, 1), jnp.float32),
        ],
    )(h, W_fc, W_attn)

    hpart = _sc_gat(srcA, dstA, sl.reshape(N), sr.reshape(N), z)

    h_out = pl.pallas_call(
        _add_body,
        grid=(10,),
        in_specs=[
            pl.BlockSpec((N // 10, D), lambda i: (i, 0)),
            pl.BlockSpec((N // 10, D), lambda i: (i, 0)),
        ],
        out_specs=pl.BlockSpec((N // 10, D), lambda i: (i, 0)),
        out_shape=jax.ShapeDtypeStruct((N, D), jnp.float32),
    )(hpart[0], hpart[1])
    return h_out


# speedup vs baseline: 1.2248x; 1.2113x over previous
import functools

import jax
import jax.numpy as jnp
from jax import lax
from jax.experimental import pallas as pl
from jax.experimental.pallas import tpu as pltpu
from jax.experimental.pallas import tpu_sc as plsc

N = 10000
E = 320000
D = 128
NC = 2
NS = 16
NW = NC * NS
NPAD = 10240
EPT = E // NW
CH = 40
NCH = EPT // CH
SCAT = 80


def _prep_body(h_ref, wfc_ref, wa_ref, z_ref, sl_ref, sr_ref, z16_ref):
    h = h_ref[...]
    z = lax.dot_general(h, wfc_ref[...], (((1,), (1,)), ((), ())),
                        preferred_element_type=jnp.float32)
    z_ref[...] = z
    a2 = wa_ref[...].reshape(2, D)
    s2 = lax.dot_general(z, a2, (((1,), (1,)), ((), ())),
                         preferred_element_type=jnp.float32)
    sl_ref[...] = s2[:, 0:1]
    sr_ref[...] = s2[:, 1:2]
    z16_ref[...] = z.astype(jnp.bfloat16)


def _add_body(a_ref, b_ref, o_ref):
    o_ref[...] = a_ref[...] + b_ref[...]


_sc_mesh = plsc.VectorSubcoreMesh(
    core_axis_name="c", subcore_axis_name="s", num_cores=NC, num_subcores=NS)


@functools.partial(
    pl.kernel,
    out_type=jax.ShapeDtypeStruct((NC, N, D), jnp.float32),
    mesh=_sc_mesh,
    compiler_params=pltpu.CompilerParams(
        needs_layout_passes=False, use_tc_tiling_on_sc=False),
    scratch_types=[
        pltpu.VMEM_SHARED((N, D), jnp.float32),
        pltpu.VMEM_SHARED((NPAD,), jnp.float32),
    ],
)
def _sc_gat(srcA_hbm, dstA_hbm, sl_hbm, sr_hbm, z_hbm, out_hbm,
            hacc_sh, den_sh):
    c = lax.axis_index("c")
    s = lax.axis_index("s")
    pl.run_scoped(
        functools.partial(_sc_gat_body, srcA_hbm, dstA_hbm, sl_hbm, sr_hbm,
                          z_hbm, out_hbm, hacc_sh, den_sh, c, s),
        pltpu.VMEM((EPT,), jnp.int32),
        pltpu.VMEM((EPT,), jnp.int32),
        pltpu.VMEM((EPT + 16,), jnp.float32),
        pltpu.SemaphoreType.DMA,
    )


def _sc_gat_body(srcA_hbm, dstA_hbm, sl_hbm, sr_hbm, z_hbm, out_hbm,
                 hacc_sh, den_sh, c, s, src_v, dst_v, ex_v, dsem):

    def phase1(sl_v, sr_v):
        pltpu.sync_copy(sl_hbm, sl_v)
        pltpu.sync_copy(sr_hbm, sr_v)

        for k in range(640 // 16):
            ex_v[pl.ds(k * 16, 16)] = jnp.zeros((16,), jnp.float32)
        pltpu.sync_copy(ex_v.at[pl.ds(0, 640)], den_sh.at[pl.ds(s * 640, 640)])

        def lane_max(acc):
            dnums = lax.GatherDimensionNumbers(
                offset_dims=(), collapsed_slice_dims=(0,), start_index_map=(0,))
            for sh in (8, 4, 2, 1):
                idx = lax.iota(jnp.int32, 16) ^ sh
                perm = lax.gather(
                    acc, idx[:, None], dnums, slice_sizes=(1,),
                    mode=lax.GatherScatterMode.PROMISE_IN_BOUNDS)
                acc = jnp.maximum(acc, perm)
            return acc

        def table_max(tab):
            def body(i, acc):
                return jnp.maximum(acc, tab[pl.ds(i * 16, 16)])
            return lane_max(lax.fori_loop(
                0, N // 16, body, jnp.full((16,), -jnp.inf, jnp.float32)))

        cv = table_max(sl_v) + table_max(sr_v)
        cv = jnp.maximum(cv, 0.01 * cv)

        plsc.subcore_barrier()

        @pl.loop(0, NC)
        def _half(q):
            b = 1 - c + q * (2 * c - 1)
            pltpu.sync_copy(srcA_hbm.at[s].at[b], src_v)
            pltpu.sync_copy(dstA_hbm.at[s].at[b], dst_v)

            @pl.loop(0, EPT // 16)
            def _edge(i):
                sidx = src_v[pl.ds(i * 16, 16)]
                didx = dst_v[pl.ds(i * 16, 16)]
                x = (plsc.load_gather(sl_v, [sidx])
                     + plsc.load_gather(sr_v, [didx]))
                e = jnp.maximum(x, 0.01 * x)
                ex_v[pl.ds(i * 16, 16)] = jnp.exp(e - cv)

            @pl.loop(0, EPT // SCAT)
            def _sc(j):
                pltpu.async_copy(
                    ex_v.at[pl.ds(j * SCAT, SCAT)],
                    den_sh.at[dst_v.at[pl.ds(j * SCAT, SCAT)]],
                    dsem, add=True)

            @pl.loop(0, EPT // SCAT)
            def _dr(j):
                pltpu.make_async_copy(
                    ex_v.at[pl.ds(0, SCAT)],
                    den_sh.at[dst_v.at[pl.ds(0, SCAT)]], dsem).wait()

        plsc.subcore_barrier()

    with jax.named_scope("p1_denom"):
        pl.run_scoped(
            phase1,
            pltpu.VMEM((N,), jnp.float32),
            pltpu.VMEM((N,), jnp.float32),
        )

    def phase2_prep(rcp_v):
        pltpu.sync_copy(den_sh.at[pl.ds(0, N)], rcp_v)

        @pl.loop(0, N // 16)
        def _rcp(i):
            v = rcp_v[pl.ds(i * 16, 16)]
            rcp_v[pl.ds(i * 16, 16)] = jnp.where(v == 0.0, 1.0, 1.0 / v)

        @pl.loop(0, EPT // 16)
        def _al(i):
            didx = dst_v[pl.ds(i * 16, 16)]
            r16 = plsc.load_gather(rcp_v, [didx])
            ex_v[pl.ds(i * 16, 16)] = ex_v[pl.ds(i * 16, 16)] * r16

    with jax.named_scope("p2_alpha"):
        pl.run_scoped(phase2_prep, pltpu.VMEM((N,), jnp.float32))

    def phase2(gA, gB, sA, sB, gsA, gsB, ssA, ssB):
        @pl.loop(0, CH)
        def _zr(r):
            for t in range(D // 16):
                sA[r, pl.ds(t * 16, 16)] = jnp.zeros((16,), jnp.float32)

        nzh = jnp.where(s == NS - 1, 10, 16)

        @pl.loop(0, nzh)
        def _zh(q):
            pltpu.sync_copy(sA, hacc_sh.at[pl.ds(s * 640 + q * CH, CH)])

        plsc.subcore_barrier()

        def gather(t, buf, sem):
            pltpu.async_copy(
                z_hbm.at[src_v.at[pl.ds(t * CH, CH)]], buf, sem)

        def scat(t, buf, sem):
            pltpu.async_copy(
                buf, hacc_sh.at[dst_v.at[pl.ds(t * CH, CH)]], sem, add=True)

        def wait_gather(t, buf, sem):
            pltpu.make_async_copy(
                z_hbm.at[src_v.at[pl.ds(t * CH, CH)]], buf, sem).wait()

        def wait_scat(t, buf, sem):
            pltpu.make_async_copy(
                buf, hacc_sh.at[dst_v.at[pl.ds(t * CH, CH)]], sem).wait()

        dnums = lax.GatherDimensionNumbers(
            offset_dims=(), collapsed_slice_dims=(0,), start_index_map=(0,))

        def scale(t, gbuf, sbuf):
            @pl.loop(0, CH // 8)
            def _grp(p):
                a16 = ex_v[pl.ds(t * CH + p * 8, 16)]
                for r in range(8):
                    av = lax.gather(
                        a16, jnp.full((16, 1), r, jnp.int32), dnums,
                        slice_sizes=(1,),
                        mode=lax.GatherScatterMode.PROMISE_IN_BOUNDS)
                    row = p * 8 + r
                    for u in range(D // 32):
                        w = plsc.bitcast(gbuf[row, pl.ds(u * 32, 32)],
                                         jnp.int32)
                        lo = plsc.bitcast(w << 16, jnp.float32)
                        hi = plsc.bitcast(w & jnp.int32(-65536), jnp.float32)
                        sbuf[row, pl.ds(u * 32, 16)] = lo * av
                        sbuf[row, pl.ds(u * 32 + 16, 16)] = hi * av

        def chunk(t, g_cur, g_oth, s_cur, gs_cur, gs_oth, ss_cur):
            wait_gather(t, g_cur, gs_cur)

            @pl.when(t + 1 < NCH)
            def _pref():
                gather(t + 1, g_oth, gs_oth)

            @pl.when(t >= 2)
            def _drain():
                wait_scat(t - 2, s_cur, ss_cur)

            scale(t, g_cur, s_cur)
            scat(t, s_cur, ss_cur)

        gather(0, gA, gsA)

        @pl.loop(0, NCH // 2)
        def _pair(i):
            chunk(2 * i, gA, gB, sA, gsA, gsB, ssA)
            chunk(2 * i + 1, gB, gA, sB, gsB, gsA, ssB)

        wait_scat(NCH - 2, sA, ssA)
        wait_scat(NCH - 1, sB, ssB)

        plsc.subcore_barrier()

        nch = jnp.where(s == NS - 1, 10, 16)

        @pl.loop(0, nch)
        def _wb(q):
            r0 = s * 640 + q * CH
            pltpu.sync_copy(hacc_sh.at[pl.ds(r0, CH)], sA)
            pltpu.sync_copy(sA, out_hbm.at[c].at[pl.ds(r0, CH)])

    with jax.named_scope("p2_main"):
        pl.run_scoped(
            phase2,
            pltpu.VMEM((CH, D), jnp.bfloat16),
            pltpu.VMEM((CH, D), jnp.bfloat16),
            pltpu.VMEM((CH, D), jnp.float32),
            pltpu.VMEM((CH, D), jnp.float32),
            pltpu.SemaphoreType.DMA,
            pltpu.SemaphoreType.DMA,
            pltpu.SemaphoreType.DMA,
            pltpu.SemaphoreType.DMA,
        )


def kernel(h, edge_index, W_fc, W_attn):
    src = edge_index[0].astype(jnp.int32)
    dst = edge_index[1].astype(jnp.int32)
    srcA = src.reshape(NS, NC, EPT)
    dstA = dst.reshape(NS, NC, EPT)

    z, sl, sr, z16 = pl.pallas_call(
        _prep_body,
        grid=(10,),
        in_specs=[
            pl.BlockSpec((N // 10, D), lambda i: (i, 0)),
            pl.BlockSpec((D, D), lambda i: (0, 0)),
            pl.BlockSpec((1, 2 * D), lambda i: (0, 0)),
        ],
        out_specs=[
            pl.BlockSpec((N // 10, D), lambda i: (i, 0)),
            pl.BlockSpec((N // 10, 1), lambda i: (i, 0)),
            pl.BlockSpec((N // 10, 1), lambda i: (i, 0)),
            pl.BlockSpec((N // 10, D), lambda i: (i, 0)),
        ],
        out_shape=[
            jax.ShapeDtypeStruct((N, D), jnp.float32),
            jax.ShapeDtypeStruct((N, 1), jnp.float32),
            jax.ShapeDtypeStruct((N, 1), jnp.float32),
            jax.ShapeDtypeStruct((N, D), jnp.bfloat16),
        ],
    )(h, W_fc, W_attn)

    z16p = z16.reshape(N, 4, 2, 16).transpose(0, 1, 3, 2).reshape(N, D)
    hpart = _sc_gat(srcA, dstA, sl.reshape(N), sr.reshape(N), z16p)

    h_out = pl.pallas_call(
        _add_body,
        grid=(10,),
        in_specs=[
            pl.BlockSpec((N // 10, D), lambda i: (i, 0)),
            pl.BlockSpec((N // 10, D), lambda i: (i, 0)),
        ],
        out_specs=pl.BlockSpec((N // 10, D), lambda i: (i, 0)),
        out_shape=jax.ShapeDtypeStruct((N, D), jnp.float32),
    )(hpart[0], hpart[1])
    return h_out
